# Initial kernel scaffold; baseline (speedup 1.0000x reference)
import jax
import jax.numpy as jnp
from jax.experimental import pallas as pl


def kernel(nodes_ids, table):
    raise NotImplementedError("write your pallas kernel here")


# speedup vs baseline: 1.8869x; 1.8869x over previous
import functools

import jax
import jax.numpy as jnp
from jax import lax
from jax.experimental import pallas as pl
from jax.experimental.pallas import tpu as pltpu
from jax.experimental.pallas import tpu_sc as plsc

VOCAB = 1000000
EMBED_DIM = 64
BATCH = 16384
HIST = 50

NUM_CORES = 2
NUM_SUBCORES = 16
NW = NUM_CORES * NUM_SUBCORES
B_TOTAL = BATCH * HIST
B_PER_W = B_TOTAL // NW
CHUNK = 800
NCHUNK = B_PER_W // CHUNK


def _gather_kernel(table_hbm, idx_hbm, out_hbm, idx_v, rows_v, gsem):
    wid = lax.axis_index("s") * NUM_CORES + lax.axis_index("c")
    base = wid * B_PER_W

    pltpu.sync_copy(idx_hbm.at[pl.ds(base, B_PER_W)], idx_v)

    def start(c, buf):
        off = pl.multiple_of(c * CHUNK, CHUNK)
        pltpu.async_copy(table_hbm.at[idx_v.at[pl.ds(off, CHUNK)]],
                         rows_v.at[buf], gsem)

    def wait(buf):
        pltpu.make_async_copy(table_hbm.at[idx_v.at[pl.ds(0, CHUNK)]],
                              rows_v.at[buf], gsem).wait()

    start(0, 0)
    start(1, 1)

    def body(g, carry):
        for b in (0, 1):
            c = g * 2 + b
            wait(b)
            pltpu.sync_copy(
                rows_v.at[b],
                out_hbm.at[pl.ds(base + pl.multiple_of(c * CHUNK, CHUNK),
                                 CHUNK)])

            @pl.when(c + 2 < NCHUNK)
            def _():
                start(c + 2, b)

        return carry

    lax.fori_loop(0, NCHUNK // 2, body, 0)


@jax.jit
def _lookup(nodes_ids, table):
    idx = nodes_ids.reshape(-1).astype(jnp.int32)
    mesh = plsc.VectorSubcoreMesh(core_axis_name="c", subcore_axis_name="s")
    out = pl.kernel(
        _gather_kernel,
        out_type=jax.ShapeDtypeStruct((B_TOTAL, EMBED_DIM), jnp.float32),
        mesh=mesh,
        scratch_types=[
            pltpu.VMEM((B_PER_W,), jnp.int32),
            pltpu.VMEM((2, CHUNK, EMBED_DIM), jnp.float32),
            pltpu.SemaphoreType.DMA,
        ],
        compiler_params=pltpu.CompilerParams(use_tc_tiling_on_sc=False),
    )(table, idx)
    return out.reshape(BATCH, HIST, EMBED_DIM)


def kernel(nodes_ids, table):
    return _lookup(nodes_ids, table)
